# Initial kernel scaffold; baseline (speedup 1.0000x reference)
import jax
import jax.numpy as jnp
from jax.experimental import pallas as pl


def kernel(x, edge_index, W1, b1, W2, b2):
    raise NotImplementedError("write your pallas kernel here")


# speedup vs baseline: 3.3374x; 3.3374x over previous
import functools

import jax
import jax.numpy as jnp
from jax import lax
from jax.experimental import pallas as pl
from jax.experimental.pallas import tpu as pltpu
from jax.experimental.pallas import tpu_sc as plsc

N = 10000
E = 160000
D = 256
NC = 2
NS = 16
NW = NC * NS
RPT = 320
NP = NW * RPT
EPT = E // NW
SUP = 5000
NSUP = E // SUP
RB = 64
QCAP = SUP + 2 * RB
_mesh = plsc.VectorSubcoreMesh(
    core_axis_name="c", subcore_axis_name="s", num_cores=NC, num_subcores=NS)


@functools.partial(
    pl.kernel,
    out_type=jax.ShapeDtypeStruct((NW * NP,), jnp.float32),
    mesh=_mesh,
    scratch_types=[
        pltpu.VMEM((EPT + 16,), jnp.int32),
        pltpu.VMEM((NP + 16,), jnp.float32),
    ],
)
def _deg_kernel(dst_hbm, out_hbm, dstage, hist):
    c = lax.axis_index("c")
    s = lax.axis_index("s")
    w = c * NS + s
    lane = lax.iota(jnp.int32, 16)
    onehot = jnp.where(lane == 0, 1.0, 0.0).astype(jnp.float32)
    trash16 = jnp.zeros((16,), jnp.int32) + (NP - 1)

    dstage[pl.ds(EPT - 8, 16)] = trash16
    pltpu.sync_copy(dst_hbm.at[pl.ds(w * EPT, EPT)],
                    dstage.at[pl.ds(0, EPT)])

    def zh(i, carry):
        hist[pl.ds(i * 16, 16)] = jnp.zeros((16,), jnp.float32)
        return carry
    lax.fori_loop(0, (NP + 16) // 16, zh, 0)

    def hbody(i, carry):
        d16 = dstage[pl.ds(i * 16, 16)]
        for e in range(16):
            d = d16[e]
            plsc.addupdate(hist.at[pl.ds(d, 16)], onehot)
        return carry
    lax.fori_loop(0, (EPT + 15) // 16, hbody, 0)

    pltpu.sync_copy(hist.at[pl.ds(0, NP)], out_hbm.at[pl.ds(w * NP, NP)])


@functools.partial(
    pl.kernel,
    out_type=jax.ShapeDtypeStruct((NP, D), jnp.float32),
    mesh=_mesh,
    scratch_types=[
        pltpu.VMEM((SUP + 16,), jnp.int32),
        pltpu.VMEM((SUP + 16,), jnp.int32),
        pltpu.VMEM((QCAP,), jnp.int32),
        pltpu.VMEM((QCAP,), jnp.int32),
        pltpu.VMEM((RB, D), jnp.float32),
        pltpu.VMEM((RPT + 8, D), jnp.float32),
        pltpu.SemaphoreType.DMA,
    ],
)
def _agg_kernel(h_hbm, src_hbm, dst_hbm, out_hbm,
                sstage, dstage, qs, qd, rows, acc, sem):
    c = lax.axis_index("c")
    s = lax.axis_index("s")
    w = c * NS + s
    lo = w * RPT
    lane = lax.iota(jnp.int32, 16)
    zero16i = jnp.zeros((16,), jnp.int32)
    r16 = lane + 1

    def zacc(i, carry):
        r = i // (D // 16)
        k = i % (D // 16)
        acc[r, pl.ds(k * 16, 16)] = jnp.zeros((16,), jnp.float32)
        return carry
    lax.fori_loop(0, (RPT + 8) * (D // 16), zacc, 0)

    sstage[pl.ds(SUP - 8, 16)] = zero16i
    dstage[pl.ds(SUP - 8, 16)] = zero16i + (NP - 1)

    def super_body(sup, carry):
        eb = sup * SUP
        pltpu.sync_copy(src_hbm.at[pl.ds(eb, SUP)], sstage.at[pl.ds(0, SUP)])
        pltpu.sync_copy(dst_hbm.at[pl.ds(eb, SUP)], dstage.at[pl.ds(0, SUP)])

        def filt(i, cnt0):
            d16 = dstage[pl.ds(i * 16, 16)]
            s16 = sstage[pl.ds(i * 16, 16)]
            inr = (d16 >= lo) & (d16 < lo + RPT)
            ld16 = d16 - lo
            inr32 = jnp.where(inr, 1, 0)
            csum = inr32
            for sh in (1, 2, 4, 8):
                shifted = jnp.take(csum, jnp.maximum(lane - sh, 0))
                csum = csum + jnp.where(lane >= sh, shifted, 0)
            idx = zero16i
            for bit in (8, 4, 2, 1):
                probe = jnp.take(csum, idx + (bit - 1))
                idx = idx + jnp.where(probe < r16, bit, 0)
            qs[pl.ds(cnt0, 16)] = jnp.take(s16, idx)
            qd[pl.ds(cnt0, 16)] = jnp.take(ld16, idx)
            return cnt0 + csum[15]
        cnt0 = lax.fori_loop(0, (SUP + 15) // 16, filt, 0)

        for t in range(RB // 16):
            pad_pos = cnt0 + lane + t * 16
            qs[pl.ds(cnt0 + t * 16, 16)] = pad_pos & 4095
            qd[pl.ds(cnt0 + t * 16, 16)] = zero16i + RPT

        nb = (cnt0 + (RB - 1)) // RB

        def drain(b, carry2):
            pltpu.async_copy(h_hbm.at[qs.at[pl.ds(b * RB, RB)]],
                             rows, sem).wait()

            def grp(g, carry3):
                ld16 = qd[pl.ds(b * RB + g * 16, 16)]
                for e in range(16):
                    ldst = ld16[e]
                    for k in range(D // 16):
                        plsc.addupdate(acc.at[ldst, pl.ds(k * 16, 16)],
                                       rows[g * 16 + e, pl.ds(k * 16, 16)])
                return carry3
            lax.fori_loop(0, RB // 16, grp, 0)
            return carry2
        lax.fori_loop(0, nb, drain, 0)
        return carry
    lax.fori_loop(0, NSUP, super_body, 0)

    pltpu.sync_copy(acc.at[pl.ds(0, RPT)], out_hbm.at[pl.ds(w * RPT, RPT)])


_BM = 400
_GRID = N // _BM


def _deg_reduce_body(hp_ref, deg_ref):
    deg_ref[...] = jnp.sum(hp_ref[...], axis=0)[:, None]


_deg_reduce = pl.pallas_call(
    _deg_reduce_body,
    grid=(NP // 512,),
    in_specs=[pl.BlockSpec((NW, 512), lambda i: (0, i))],
    out_specs=pl.BlockSpec((512, 1), lambda i: (i, 0)),
    out_shape=jax.ShapeDtypeStruct((NP, 1), jnp.float32),
)


def _mm_scale_body(x_ref, w_ref, deg_ref, h_ref, dinv_ref):
    dv = lax.rsqrt(deg_ref[...] + 1.0)
    h = jnp.dot(x_ref[...], w_ref[...], preferred_element_type=jnp.float32)
    h_ref[...] = h * dv
    dinv_ref[...] = dv


def _layer2_body(acc_ref, h1_ref, dinv_ref, b1_ref, w2_ref, h2_ref):
    dv = dinv_ref[...]
    z = jnp.maximum((acc_ref[...] + h1_ref[...]) * dv + b1_ref[...], 0.0)
    h2_ref[...] = jnp.dot(z, w2_ref[...],
                          preferred_element_type=jnp.float32) * dv


def _final_body(acc_ref, h2_ref, dinv_ref, b2_ref, out_ref):
    out_ref[...] = ((acc_ref[...] + h2_ref[...]) * dinv_ref[...]
                    + b2_ref[...])


_row_spec = pl.BlockSpec((_BM, D), lambda i: (i, 0))
_col_spec = pl.BlockSpec((_BM, 1), lambda i: (i, 0))
_w_spec = pl.BlockSpec((D, D), lambda i: (0, 0))
_b_spec = pl.BlockSpec((1, D), lambda i: (0, 0))

_mm_scale = pl.pallas_call(
    _mm_scale_body,
    grid=(_GRID,),
    in_specs=[_row_spec, _w_spec, _col_spec],
    out_specs=[_row_spec, _col_spec],
    out_shape=[jax.ShapeDtypeStruct((N, D), jnp.float32),
               jax.ShapeDtypeStruct((N, 1), jnp.float32)],
)

_layer2 = pl.pallas_call(
    _layer2_body,
    grid=(_GRID,),
    in_specs=[_row_spec, _row_spec, _col_spec, _b_spec, _w_spec],
    out_specs=_row_spec,
    out_shape=jax.ShapeDtypeStruct((N, D), jnp.float32),
)

_final = pl.pallas_call(
    _final_body,
    grid=(_GRID,),
    in_specs=[_row_spec, _row_spec, _col_spec, _b_spec],
    out_specs=_row_spec,
    out_shape=jax.ShapeDtypeStruct((N, D), jnp.float32),
)


def kernel(x, edge_index, W1, b1, W2, b2):
    src = edge_index[0]
    dst = edge_index[1]

    degp = _deg_kernel(dst).reshape(NW, NP)
    deg = _deg_reduce(degp)[:N]
    h1, dinv = _mm_scale(x, W1, deg)
    acc1 = _agg_kernel(h1, src, dst)[:N]
    h2 = _layer2(acc1, h1, dinv, b1.reshape(1, D), W2)
    acc2 = _agg_kernel(h2, src, dst)[:N]
    out = _final(acc2, h2, dinv, b2.reshape(1, D))
    return out


# speedup vs baseline: 3.7196x; 1.1145x over previous
import functools

import jax
import jax.numpy as jnp
from jax import lax
from jax.experimental import pallas as pl
from jax.experimental.pallas import tpu as pltpu
from jax.experimental.pallas import tpu_sc as plsc

N = 10000
E = 160000
D = 256
NC = 2
NS = 16
NW = NC * NS
RPT = 320
NP = NW * RPT
EPT = E // NW
SUP = 5000
NSUP = E // SUP
RB = 64
QCAP = 5136
SUPR = 5120
_mesh = plsc.VectorSubcoreMesh(
    core_axis_name="c", subcore_axis_name="s", num_cores=NC, num_subcores=NS)


@functools.partial(
    pl.kernel,
    out_type=jax.ShapeDtypeStruct((NW * NP,), jnp.float32),
    mesh=_mesh,
    scratch_types=[
        pltpu.VMEM((EPT + 16,), jnp.int32),
        pltpu.VMEM((NP + 16,), jnp.float32),
    ],
)
def _deg_kernel(dst_hbm, out_hbm, dstage, hist):
    c = lax.axis_index("c")
    s = lax.axis_index("s")
    w = c * NS + s
    lane = lax.iota(jnp.int32, 16)
    onehot = jnp.where(lane == 0, 1.0, 0.0).astype(jnp.float32)
    trash16 = jnp.zeros((16,), jnp.int32) + (NP - 1)

    dstage[pl.ds(EPT - 8, 16)] = trash16
    pltpu.sync_copy(dst_hbm.at[pl.ds(w * EPT, EPT)],
                    dstage.at[pl.ds(0, EPT)])

    def zh(i, carry):
        hist[pl.ds(i * 16, 16)] = jnp.zeros((16,), jnp.float32)
        return carry
    lax.fori_loop(0, (NP + 16) // 16, zh, 0)

    def hbody(i, carry):
        d16 = dstage[pl.ds(i * 16, 16)]
        for e in range(16):
            d = d16[e]
            plsc.addupdate(hist.at[pl.ds(d, 16)], onehot)
        return carry
    lax.fori_loop(0, (EPT + 15) // 16, hbody, 0)

    pltpu.sync_copy(hist.at[pl.ds(0, NP)], out_hbm.at[pl.ds(w * NP, NP)])


@functools.partial(
    pl.kernel,
    out_type=[jax.ShapeDtypeStruct((NW * NSUP * SUPR,), jnp.int32),
              jax.ShapeDtypeStruct((NW * NSUP * SUPR,), jnp.int32),
              jax.ShapeDtypeStruct((NW * NSUP,), jnp.int32)],
    mesh=_mesh,
    scratch_types=[
        pltpu.VMEM((SUP + 16,), jnp.int32),
        pltpu.VMEM((SUP + 16,), jnp.int32),
        pltpu.VMEM((QCAP,), jnp.int32),
        pltpu.VMEM((QCAP,), jnp.int32),
        pltpu.VMEM((NSUP + 16,), jnp.int32),
    ],
)
def _prep_kernel(src_hbm, dst_hbm, qs_hbm, qd_hbm, cnt_hbm,
                 sstage, dstage, qs, qd, cvec):
    c = lax.axis_index("c")
    s = lax.axis_index("s")
    w = c * NS + s
    lo = w * RPT
    lane = lax.iota(jnp.int32, 16)
    zero16i = jnp.zeros((16,), jnp.int32)
    r16 = lane + 1

    sstage[pl.ds(SUP - 8, 16)] = zero16i
    dstage[pl.ds(SUP - 8, 16)] = zero16i + (NP - 1)

    def super_body(sup, carry):
        eb = sup * SUP
        pltpu.sync_copy(src_hbm.at[pl.ds(eb, SUP)], sstage.at[pl.ds(0, SUP)])
        pltpu.sync_copy(dst_hbm.at[pl.ds(eb, SUP)], dstage.at[pl.ds(0, SUP)])

        def filt(i, cnt0):
            d16 = dstage[pl.ds(i * 16, 16)]
            s16 = sstage[pl.ds(i * 16, 16)]
            inr = (d16 >= lo) & (d16 < lo + RPT)
            ld16 = d16 - lo
            inr32 = jnp.where(inr, 1, 0)
            csum = inr32
            for sh in (1, 2, 4, 8):
                shifted = jnp.take(csum, jnp.maximum(lane - sh, 0))
                csum = csum + jnp.where(lane >= sh, shifted, 0)
            idx = zero16i
            for bit in (8, 4, 2, 1):
                probe = jnp.take(csum, idx + (bit - 1))
                idx = idx + jnp.where(probe < r16, bit, 0)
            qs[pl.ds(cnt0, 16)] = jnp.take(s16, idx)
            qd[pl.ds(cnt0, 16)] = jnp.take(ld16, idx)
            return cnt0 + csum[15]
        cnt0 = lax.fori_loop(0, (SUP + 15) // 16, filt, 0)

        for t in range(RB // 16):
            pad_pos = cnt0 + lane + t * 16
            qs[pl.ds(cnt0 + t * 16, 16)] = pad_pos & 4095
            qd[pl.ds(cnt0 + t * 16, 16)] = zero16i + RPT

        cvec[pl.ds(sup, 16)] = zero16i + cnt0

        base = (w * NSUP + sup) * SUPR
        nw256 = (cnt0 + RB + 255) // 256

        def wout(bq, carry2):
            pltpu.sync_copy(qs.at[pl.ds(bq * 256, 256)],
                            qs_hbm.at[pl.ds(base + bq * 256, 256)])
            pltpu.sync_copy(qd.at[pl.ds(bq * 256, 256)],
                            qd_hbm.at[pl.ds(base + bq * 256, 256)])
            return carry2
        lax.fori_loop(0, nw256, wout, 0)
        return carry
    lax.fori_loop(0, NSUP, super_body, 0)

    pltpu.sync_copy(cvec.at[pl.ds(0, NSUP)], cnt_hbm.at[pl.ds(w * NSUP, NSUP)])


@functools.partial(
    pl.kernel,
    out_type=jax.ShapeDtypeStruct((NP, D), jnp.float32),
    mesh=_mesh,
    scratch_types=[
        pltpu.VMEM((NSUP + 16,), jnp.int32),
        pltpu.VMEM((RB,), jnp.int32),
        pltpu.VMEM((RB,), jnp.int32),
        pltpu.VMEM((RB, D), jnp.float32),
        pltpu.VMEM((RPT + 8, D), jnp.float32),
        pltpu.SemaphoreType.DMA,
    ],
)
def _agg_kernel(h_hbm, qs_hbm, qd_hbm, cnt_hbm, out_hbm,
                cstage, qsb, qdb, rows, acc, sem):
    c = lax.axis_index("c")
    s = lax.axis_index("s")
    w = c * NS + s

    def zacc(i, carry):
        r = i // (D // 16)
        k = i % (D // 16)
        acc[r, pl.ds(k * 16, 16)] = jnp.zeros((16,), jnp.float32)
        return carry
    lax.fori_loop(0, (RPT + 8) * (D // 16), zacc, 0)

    pltpu.sync_copy(cnt_hbm.at[pl.ds(w * NSUP, NSUP)],
                    cstage.at[pl.ds(0, NSUP)])

    def super_body(sup, carry):
        cnt0 = cstage[pl.ds(sup, 16)][0]
        nb = (cnt0 + (RB - 1)) // RB
        base = (w * NSUP + sup) * SUPR

        def drain(b, carry2):
            pltpu.sync_copy(qs_hbm.at[pl.ds(base + b * RB, RB)], qsb)
            pltpu.sync_copy(qd_hbm.at[pl.ds(base + b * RB, RB)], qdb)
            pltpu.async_copy(h_hbm.at[qsb], rows, sem).wait()

            def grp(g, carry3):
                ld16 = qdb[pl.ds(g * 16, 16)]
                for e in range(16):
                    ldst = ld16[e]
                    for k in range(D // 16):
                        plsc.addupdate(acc.at[ldst, pl.ds(k * 16, 16)],
                                       rows[g * 16 + e, pl.ds(k * 16, 16)])
                return carry3
            lax.fori_loop(0, RB // 16, grp, 0)
            return carry2
        lax.fori_loop(0, nb, drain, 0)
        return carry
    lax.fori_loop(0, NSUP, super_body, 0)

    pltpu.sync_copy(acc.at[pl.ds(0, RPT)], out_hbm.at[pl.ds(w * RPT, RPT)])


_BM = 400
_GRID = N // _BM


def _deg_reduce_body(hp_ref, deg_ref):
    deg_ref[...] = jnp.sum(hp_ref[...], axis=0)[:, None]


_deg_reduce = pl.pallas_call(
    _deg_reduce_body,
    grid=(NP // 512,),
    in_specs=[pl.BlockSpec((NW, 512), lambda i: (0, i))],
    out_specs=pl.BlockSpec((512, 1), lambda i: (i, 0)),
    out_shape=jax.ShapeDtypeStruct((NP, 1), jnp.float32),
)


def _mm_scale_body(x_ref, w_ref, deg_ref, h_ref, dinv_ref):
    dv = lax.rsqrt(deg_ref[...] + 1.0)
    h = jnp.dot(x_ref[...], w_ref[...], preferred_element_type=jnp.float32)
    h_ref[...] = h * dv
    dinv_ref[...] = dv


def _layer2_body(acc_ref, h1_ref, dinv_ref, b1_ref, w2_ref, h2_ref):
    dv = dinv_ref[...]
    z = jnp.maximum((acc_ref[...] + h1_ref[...]) * dv + b1_ref[...], 0.0)
    h2_ref[...] = jnp.dot(z, w2_ref[...],
                          preferred_element_type=jnp.float32) * dv


def _final_body(acc_ref, h2_ref, dinv_ref, b2_ref, out_ref):
    out_ref[...] = ((acc_ref[...] + h2_ref[...]) * dinv_ref[...]
                    + b2_ref[...])


_row_spec = pl.BlockSpec((_BM, D), lambda i: (i, 0))
_col_spec = pl.BlockSpec((_BM, 1), lambda i: (i, 0))
_w_spec = pl.BlockSpec((D, D), lambda i: (0, 0))
_b_spec = pl.BlockSpec((1, D), lambda i: (0, 0))

_mm_scale = pl.pallas_call(
    _mm_scale_body,
    grid=(_GRID,),
    in_specs=[_row_spec, _w_spec, _col_spec],
    out_specs=[_row_spec, _col_spec],
    out_shape=[jax.ShapeDtypeStruct((N, D), jnp.float32),
               jax.ShapeDtypeStruct((N, 1), jnp.float32)],
)

_layer2 = pl.pallas_call(
    _layer2_body,
    grid=(_GRID,),
    in_specs=[_row_spec, _row_spec, _col_spec, _b_spec, _w_spec],
    out_specs=_row_spec,
    out_shape=jax.ShapeDtypeStruct((N, D), jnp.float32),
)

_final = pl.pallas_call(
    _final_body,
    grid=(_GRID,),
    in_specs=[_row_spec, _row_spec, _col_spec, _b_spec],
    out_specs=_row_spec,
    out_shape=jax.ShapeDtypeStruct((N, D), jnp.float32),
)


def kernel(x, edge_index, W1, b1, W2, b2):
    src = edge_index[0]
    dst = edge_index[1]

    degp = _deg_kernel(dst).reshape(NW, NP)
    deg = _deg_reduce(degp)[:N]
    qs_h, qd_h, cnt_h = _prep_kernel(src, dst)
    h1, dinv = _mm_scale(x, W1, deg)
    acc1 = _agg_kernel(h1, qs_h, qd_h, cnt_h)[:N]
    h2 = _layer2(acc1, h1, dinv, b1.reshape(1, D), W2)
    acc2 = _agg_kernel(h2, qs_h, qd_h, cnt_h)[:N]
    out = _final(acc2, h2, dinv, b2.reshape(1, D))
    return out


# speedup vs baseline: 4.5078x; 1.2119x over previous
import functools

import jax
import jax.numpy as jnp
from jax import lax
from jax.experimental import pallas as pl
from jax.experimental.pallas import tpu as pltpu
from jax.experimental.pallas import tpu_sc as plsc

N = 10000
E = 160000
D = 256
NC = 2
NS = 16
NW = NC * NS
RPT = 320
NP = NW * RPT
EPT = E // NW
SUP = 20000
NSUP = E // SUP
RB = 64
QCAP = 20480
SUPR = 20480
CH = 2048
_mesh = plsc.VectorSubcoreMesh(
    core_axis_name="c", subcore_axis_name="s", num_cores=NC, num_subcores=NS)


@functools.partial(
    pl.kernel,
    out_type=jax.ShapeDtypeStruct((NW * NP,), jnp.float32),
    mesh=_mesh,
    scratch_types=[
        pltpu.VMEM((EPT + 16,), jnp.int32),
        pltpu.VMEM((NP + 16,), jnp.float32),
    ],
)
def _deg_kernel(dst_hbm, out_hbm, dstage, hist):
    c = lax.axis_index("c")
    s = lax.axis_index("s")
    w = c * NS + s
    lane = lax.iota(jnp.int32, 16)
    onehot = jnp.where(lane == 0, 1.0, 0.0).astype(jnp.float32)
    trash16 = jnp.zeros((16,), jnp.int32) + (NP - 1)

    dstage[pl.ds(EPT - 8, 16)] = trash16
    pltpu.sync_copy(dst_hbm.at[pl.ds(w * EPT, EPT)],
                    dstage.at[pl.ds(0, EPT)])

    def zh(i, carry):
        hist[pl.ds(i * 16, 16)] = jnp.zeros((16,), jnp.float32)
        return carry
    lax.fori_loop(0, (NP + 16) // 16, zh, 0)

    def hbody(i, carry):
        d16 = dstage[pl.ds(i * 16, 16)]
        for e in range(16):
            d = d16[e]
            plsc.addupdate(hist.at[pl.ds(d, 16)], onehot)
        return carry
    lax.fori_loop(0, (EPT + 15) // 16, hbody, 0)

    pltpu.sync_copy(hist.at[pl.ds(0, NP)], out_hbm.at[pl.ds(w * NP, NP)])


@functools.partial(
    pl.kernel,
    out_type=[jax.ShapeDtypeStruct((NW * NSUP * SUPR,), jnp.int32),
              jax.ShapeDtypeStruct((NW * NSUP,), jnp.int32)],
    mesh=_mesh,
    scratch_types=[
        pltpu.VMEM((SUP + 16,), jnp.int32),
        pltpu.VMEM((SUP + 16,), jnp.int32),
        pltpu.VMEM((QCAP,), jnp.int32),
        pltpu.VMEM((NSUP + 16,), jnp.int32),
    ],
)
def _prep_kernel(src_hbm, dst_hbm, qp_hbm, cnt_hbm,
                 sstage, dstage, qp, cvec):
    c = lax.axis_index("c")
    s = lax.axis_index("s")
    w = c * NS + s
    lo = w * RPT
    lane = lax.iota(jnp.int32, 16)
    zero16i = jnp.zeros((16,), jnp.int32)
    r16 = lane + 1

    sstage[pl.ds(SUP - 8, 16)] = zero16i
    dstage[pl.ds(SUP - 8, 16)] = zero16i + (NP - 1)

    def super_body(sup, carry):
        eb = sup * SUP
        pltpu.sync_copy(src_hbm.at[pl.ds(eb, SUP)], sstage.at[pl.ds(0, SUP)])
        pltpu.sync_copy(dst_hbm.at[pl.ds(eb, SUP)], dstage.at[pl.ds(0, SUP)])

        def filt(i, cnt0):
            d16 = dstage[pl.ds(i * 16, 16)]
            s16 = sstage[pl.ds(i * 16, 16)]
            inr = (d16 >= lo) & (d16 < lo + RPT)
            ld16 = d16 - lo
            inr32 = jnp.where(inr, 1, 0)
            csum = inr32
            for sh in (1, 2, 4, 8):
                shifted = jnp.take(csum, jnp.maximum(lane - sh, 0))
                csum = csum + jnp.where(lane >= sh, shifted, 0)
            idx = zero16i
            for bit in (8, 4, 2, 1):
                probe = jnp.take(csum, idx + (bit - 1))
                idx = idx + jnp.where(probe < r16, bit, 0)
            p16 = s16 * 512 + jnp.where(inr, ld16, RPT)
            qp[pl.ds(cnt0, 16)] = jnp.take(p16, idx)
            return cnt0 + csum[15]
        cnt0 = lax.fori_loop(0, (SUP + 15) // 16, filt, 0)

        for t in range(RB // 16):
            pad_pos = cnt0 + lane + t * 16
            qp[pl.ds(cnt0 + t * 16, 16)] = ((pad_pos & 4095) * 512 + RPT)

        cvec[pl.ds(sup, 16)] = zero16i + cnt0

        base = (w * NSUP + sup) * SUPR
        nw = (cnt0 + RB + 1023) // 1024

        def wout(bq, carry2):
            pltpu.sync_copy(qp.at[pl.ds(bq * 1024, 1024)],
                            qp_hbm.at[pl.ds(base + bq * 1024, 1024)])
            return carry2
        lax.fori_loop(0, nw, wout, 0)
        return carry
    lax.fori_loop(0, NSUP, super_body, 0)

    pltpu.sync_copy(cvec.at[pl.ds(0, NSUP)], cnt_hbm.at[pl.ds(w * NSUP, NSUP)])


@functools.partial(
    pl.kernel,
    out_type=jax.ShapeDtypeStruct((NP, D), jnp.float32),
    mesh=_mesh,
    scratch_types=[
        pltpu.VMEM((NSUP + 16,), jnp.int32),
        pltpu.VMEM((CH,), jnp.int32),
        pltpu.VMEM((CH,), jnp.int32),
        pltpu.VMEM((CH,), jnp.int32),
        pltpu.VMEM((RB, D), jnp.float32),
        pltpu.VMEM((RPT + 8, D), jnp.float32),
        pltpu.SemaphoreType.DMA,
    ],
)
def _agg_kernel(h_hbm, qp_hbm, cnt_hbm, out_hbm,
                cstage, pstage, qsb, qdb, rows, acc, sem):
    c = lax.axis_index("c")
    s = lax.axis_index("s")
    w = c * NS + s

    def zacc(i, carry):
        r = i // (D // 16)
        k = i % (D // 16)
        acc[r, pl.ds(k * 16, 16)] = jnp.zeros((16,), jnp.float32)
        return carry
    lax.fori_loop(0, (RPT + 8) * (D // 16), zacc, 0)

    pltpu.sync_copy(cnt_hbm.at[pl.ds(w * NSUP, NSUP)],
                    cstage.at[pl.ds(0, NSUP)])

    def super_body(sup, carry):
        cnt0 = cstage[pl.ds(sup, 16)][0]
        nb = (cnt0 + (RB - 1)) // RB
        nch = (nb * RB + (CH - 1)) // CH
        base = (w * NSUP + sup) * SUPR

        def chunk(ci, carry1):
            pltpu.sync_copy(qp_hbm.at[pl.ds(base + ci * CH, CH)], pstage)

            def unpack(u, carry1b):
                p16 = pstage[pl.ds(u * 16, 16)]
                qsb[pl.ds(u * 16, 16)] = p16 >> 9
                qdb[pl.ds(u * 16, 16)] = p16 & 511
                return carry1b
            lax.fori_loop(0, CH // 16, unpack, 0)
            nbc = jnp.minimum(nb - ci * (CH // RB), CH // RB)

            def drain(b, carry2):
                pltpu.async_copy(h_hbm.at[qsb.at[pl.ds(b * RB, RB)]],
                                 rows, sem).wait()

                def grp(g, carry3):
                    ld16 = qdb[pl.ds(b * RB + g * 16, 16)]
                    for e in range(16):
                        ldst = ld16[e]
                        for k in range(D // 16):
                            plsc.addupdate(acc.at[ldst, pl.ds(k * 16, 16)],
                                           rows[g * 16 + e, pl.ds(k * 16, 16)])
                    return carry3
                lax.fori_loop(0, RB // 16, grp, 0)
                return carry2
            lax.fori_loop(0, nbc, drain, 0)
            return carry1
        lax.fori_loop(0, nch, chunk, 0)
        return carry
    lax.fori_loop(0, NSUP, super_body, 0)

    pltpu.sync_copy(acc.at[pl.ds(0, RPT)], out_hbm.at[pl.ds(w * RPT, RPT)])


_BM = 400
_GRID = N // _BM


def _deg_reduce_body(hp_ref, deg_ref):
    deg_ref[...] = jnp.sum(hp_ref[...], axis=0)[:, None]


_deg_reduce = pl.pallas_call(
    _deg_reduce_body,
    grid=(NP // 512,),
    in_specs=[pl.BlockSpec((NW, 512), lambda i: (0, i))],
    out_specs=pl.BlockSpec((512, 1), lambda i: (i, 0)),
    out_shape=jax.ShapeDtypeStruct((NP, 1), jnp.float32),
)


def _mm_scale_body(x_ref, w_ref, deg_ref, h_ref, dinv_ref):
    dv = lax.rsqrt(deg_ref[...] + 1.0)
    h = jnp.dot(x_ref[...], w_ref[...], preferred_element_type=jnp.float32)
    h_ref[...] = h * dv
    dinv_ref[...] = dv


def _layer2_body(acc_ref, h1_ref, dinv_ref, b1_ref, w2_ref, h2_ref):
    dv = dinv_ref[...]
    z = jnp.maximum((acc_ref[...] + h1_ref[...]) * dv + b1_ref[...], 0.0)
    h2_ref[...] = jnp.dot(z, w2_ref[...],
                          preferred_element_type=jnp.float32) * dv


def _final_body(acc_ref, h2_ref, dinv_ref, b2_ref, out_ref):
    out_ref[...] = ((acc_ref[...] + h2_ref[...]) * dinv_ref[...]
                    + b2_ref[...])


_row_spec = pl.BlockSpec((_BM, D), lambda i: (i, 0))
_col_spec = pl.BlockSpec((_BM, 1), lambda i: (i, 0))
_w_spec = pl.BlockSpec((D, D), lambda i: (0, 0))
_b_spec = pl.BlockSpec((1, D), lambda i: (0, 0))

_mm_scale = pl.pallas_call(
    _mm_scale_body,
    grid=(_GRID,),
    in_specs=[_row_spec, _w_spec, _col_spec],
    out_specs=[_row_spec, _col_spec],
    out_shape=[jax.ShapeDtypeStruct((N, D), jnp.float32),
               jax.ShapeDtypeStruct((N, 1), jnp.float32)],
)

_layer2 = pl.pallas_call(
    _layer2_body,
    grid=(_GRID,),
    in_specs=[_row_spec, _row_spec, _col_spec, _b_spec, _w_spec],
    out_specs=_row_spec,
    out_shape=jax.ShapeDtypeStruct((N, D), jnp.float32),
)

_final = pl.pallas_call(
    _final_body,
    grid=(_GRID,),
    in_specs=[_row_spec, _row_spec, _col_spec, _b_spec],
    out_specs=_row_spec,
    out_shape=jax.ShapeDtypeStruct((N, D), jnp.float32),
)


def kernel(x, edge_index, W1, b1, W2, b2):
    src = edge_index[0]
    dst = edge_index[1]

    degp = _deg_kernel(dst).reshape(NW, NP)
    deg = _deg_reduce(degp)[:N]
    qp_h, cnt_h = _prep_kernel(src, dst)
    h1, dinv = _mm_scale(x, W1, deg)
    acc1 = _agg_kernel(h1, qp_h, cnt_h)[:N]
    h2 = _layer2(acc1, h1, dinv, b1.reshape(1, D), W2)
    acc2 = _agg_kernel(h2, qp_h, cnt_h)[:N]
    out = _final(acc2, h2, dinv, b2.reshape(1, D))
    return out


# speedup vs baseline: 5.2466x; 1.1639x over previous
import functools

import jax
import jax.numpy as jnp
from jax import lax
from jax.experimental import pallas as pl
from jax.experimental.pallas import tpu as pltpu
from jax.experimental.pallas import tpu_sc as plsc

N = 10000
E = 160000
D = 256
NC = 2
NS = 16
NW = NC * NS
RPT = 320
NP = NW * RPT
EPT = E // NW
SUP = 20000
NSUP = E // SUP
RB = 64
QCAP = 20480
SUPR = 20480
CH = 2048
_mesh = plsc.VectorSubcoreMesh(
    core_axis_name="c", subcore_axis_name="s", num_cores=NC, num_subcores=NS)


@functools.partial(
    pl.kernel,
    out_type=jax.ShapeDtypeStruct((NW * NP,), jnp.float32),
    mesh=_mesh,
    scratch_types=[
        pltpu.VMEM((EPT + 16,), jnp.int32),
        pltpu.VMEM((NP + 16,), jnp.float32),
    ],
)
def _deg_kernel(dst_hbm, out_hbm, dstage, hist):
    c = lax.axis_index("c")
    s = lax.axis_index("s")
    w = c * NS + s
    lane = lax.iota(jnp.int32, 16)
    onehot = jnp.where(lane == 0, 1.0, 0.0).astype(jnp.float32)
    trash16 = jnp.zeros((16,), jnp.int32) + (NP - 1)

    dstage[pl.ds(EPT - 8, 16)] = trash16
    pltpu.sync_copy(dst_hbm.at[pl.ds(w * EPT, EPT)],
                    dstage.at[pl.ds(0, EPT)])

    def zh(i, carry):
        hist[pl.ds(i * 16, 16)] = jnp.zeros((16,), jnp.float32)
        return carry
    lax.fori_loop(0, (NP + 16) // 16, zh, 0)

    def hbody(i, carry):
        d16 = dstage[pl.ds(i * 16, 16)]
        for e in range(16):
            d = d16[e]
            plsc.addupdate(hist.at[pl.ds(d, 16)], onehot)
        return carry
    lax.fori_loop(0, (EPT + 15) // 16, hbody, 0)

    pltpu.sync_copy(hist.at[pl.ds(0, NP)], out_hbm.at[pl.ds(w * NP, NP)])


@functools.partial(
    pl.kernel,
    out_type=[jax.ShapeDtypeStruct((NW * NSUP * SUPR,), jnp.int32),
              jax.ShapeDtypeStruct((NW * NSUP,), jnp.int32)],
    mesh=_mesh,
    scratch_types=[
        pltpu.VMEM((SUP + 16,), jnp.int32),
        pltpu.VMEM((SUP + 16,), jnp.int32),
        pltpu.VMEM((QCAP,), jnp.int32),
        pltpu.VMEM((NSUP + 16,), jnp.int32),
    ],
)
def _prep_kernel(src_hbm, dst_hbm, qp_hbm, cnt_hbm,
                 sstage, dstage, qp, cvec):
    c = lax.axis_index("c")
    s = lax.axis_index("s")
    w = c * NS + s
    lo = w * RPT
    lane = lax.iota(jnp.int32, 16)
    zero16i = jnp.zeros((16,), jnp.int32)
    r16 = lane + 1

    sstage[pl.ds(SUP - 8, 16)] = zero16i
    dstage[pl.ds(SUP - 8, 16)] = zero16i + (NP - 1)

    def super_body(sup, carry):
        eb = sup * SUP
        pltpu.sync_copy(src_hbm.at[pl.ds(eb, SUP)], sstage.at[pl.ds(0, SUP)])
        pltpu.sync_copy(dst_hbm.at[pl.ds(eb, SUP)], dstage.at[pl.ds(0, SUP)])

        def filt(i, cnt0):
            d16 = dstage[pl.ds(i * 16, 16)]
            s16 = sstage[pl.ds(i * 16, 16)]
            inr = (d16 >= lo) & (d16 < lo + RPT)
            ld16 = d16 - lo
            inr32 = jnp.where(inr, 1, 0)
            csum = inr32
            for sh in (1, 2, 4, 8):
                shifted = jnp.take(csum, jnp.maximum(lane - sh, 0))
                csum = csum + jnp.where(lane >= sh, shifted, 0)
            idx = zero16i
            for bit in (8, 4, 2, 1):
                probe = jnp.take(csum, idx + (bit - 1))
                idx = idx + jnp.where(probe < r16, bit, 0)
            p16 = s16 * 512 + jnp.where(inr, ld16, RPT)
            qp[pl.ds(cnt0, 16)] = jnp.take(p16, idx)
            return cnt0 + csum[15]
        cnt0 = lax.fori_loop(0, (SUP + 15) // 16, filt, 0)

        for t in range(RB // 16):
            pad_pos = cnt0 + lane + t * 16
            qp[pl.ds(cnt0 + t * 16, 16)] = ((pad_pos & 4095) * 512 + RPT)

        cvec[pl.ds(sup, 16)] = zero16i + cnt0

        base = (w * NSUP + sup) * SUPR
        nw = (cnt0 + RB + 1023) // 1024

        def wout(bq, carry2):
            pltpu.sync_copy(qp.at[pl.ds(bq * 1024, 1024)],
                            qp_hbm.at[pl.ds(base + bq * 1024, 1024)])
            return carry2
        lax.fori_loop(0, nw, wout, 0)
        return carry
    lax.fori_loop(0, NSUP, super_body, 0)

    pltpu.sync_copy(cvec.at[pl.ds(0, NSUP)], cnt_hbm.at[pl.ds(w * NSUP, NSUP)])


@functools.partial(
    pl.kernel,
    out_type=jax.ShapeDtypeStruct((NP, D), jnp.float32),
    mesh=_mesh,
    scratch_types=[
        pltpu.VMEM((NSUP + 16,), jnp.int32),
        pltpu.VMEM((CH,), jnp.int32),
        pltpu.VMEM((CH,), jnp.int32),
        pltpu.VMEM((CH,), jnp.int32),
        pltpu.VMEM((2, RB, D), jnp.float32),
        pltpu.VMEM((RPT + 8, D), jnp.float32),
        pltpu.SemaphoreType.DMA,
        pltpu.SemaphoreType.DMA,
    ],
)
def _agg_kernel(h_hbm, qp_hbm, cnt_hbm, out_hbm,
                cstage, pstage, qsb, qdb, rows, acc, sem0, sem1):
    c = lax.axis_index("c")
    s = lax.axis_index("s")
    w = c * NS + s

    def zacc(i, carry):
        r = i // (D // 16)
        k = i % (D // 16)
        acc[r, pl.ds(k * 16, 16)] = jnp.zeros((16,), jnp.float32)
        return carry
    lax.fori_loop(0, (RPT + 8) * (D // 16), zacc, 0)

    pltpu.sync_copy(cnt_hbm.at[pl.ds(w * NSUP, NSUP)],
                    cstage.at[pl.ds(0, NSUP)])

    def super_body(sup, carry):
        cnt0 = cstage[pl.ds(sup, 16)][0]
        nb = (cnt0 + (RB - 1)) // RB
        nch = (nb * RB + (CH - 1)) // CH
        base = (w * NSUP + sup) * SUPR

        def chunk(ci, carry1):
            pltpu.sync_copy(qp_hbm.at[pl.ds(base + ci * CH, CH)], pstage)

            def unpack(u, carry1b):
                p16 = pstage[pl.ds(u * 16, 16)]
                qsb[pl.ds(u * 16, 16)] = p16 >> 9
                qdb[pl.ds(u * 16, 16)] = p16 & 511
                return carry1b
            lax.fori_loop(0, CH // 16, unpack, 0)
            nbc = jnp.minimum(nb - ci * (CH // RB), CH // RB)

            def issue(b, buf, sem):
                pltpu.async_copy(h_hbm.at[qsb.at[pl.ds(b * RB, RB)]],
                                 rows.at[buf], sem)

            def wait(buf, sem):
                pltpu.make_async_copy(h_hbm.at[qsb.at[pl.ds(0, RB)]],
                                      rows.at[buf], sem).wait()

            def rmw(b, buf):
                def grp(g, carry3):
                    ld16 = qdb[pl.ds(b * RB + g * 16, 16)]
                    for e in range(16):
                        ldst = ld16[e]
                        for k in range(D // 16):
                            plsc.addupdate(
                                acc.at[ldst, pl.ds(k * 16, 16)],
                                rows[buf, g * 16 + e, pl.ds(k * 16, 16)])
                    return carry3
                lax.fori_loop(0, RB // 16, grp, 0)

            @pl.when(nbc > 0)
            def _():
                issue(0, 0, sem0)

            def pair(jj, carry2):
                b0 = jj * 2
                b1 = b0 + 1

                @pl.when(b1 < nbc)
                def _():
                    issue(b1, 1, sem1)
                wait(0, sem0)
                rmw(b0, 0)

                @pl.when(b0 + 2 < nbc)
                def _():
                    issue(b0 + 2, 0, sem0)

                @pl.when(b1 < nbc)
                def _():
                    wait(1, sem1)
                    rmw(b1, 1)
                return carry2
            lax.fori_loop(0, (nbc + 1) // 2, pair, 0)
            return carry1
        lax.fori_loop(0, nch, chunk, 0)
        return carry
    lax.fori_loop(0, NSUP, super_body, 0)

    pltpu.sync_copy(acc.at[pl.ds(0, RPT)], out_hbm.at[pl.ds(w * RPT, RPT)])


_BM = 400
_GRID = N // _BM


def _deg_reduce_body(hp_ref, deg_ref):
    deg_ref[...] = jnp.sum(hp_ref[...], axis=0)[:, None]


_deg_reduce = pl.pallas_call(
    _deg_reduce_body,
    grid=(NP // 512,),
    in_specs=[pl.BlockSpec((NW, 512), lambda i: (0, i))],
    out_specs=pl.BlockSpec((512, 1), lambda i: (i, 0)),
    out_shape=jax.ShapeDtypeStruct((NP, 1), jnp.float32),
)


def _mm_scale_body(x_ref, w_ref, deg_ref, h_ref, dinv_ref):
    dv = lax.rsqrt(deg_ref[...] + 1.0)
    h = jnp.dot(x_ref[...], w_ref[...], preferred_element_type=jnp.float32)
    h_ref[...] = h * dv
    dinv_ref[...] = dv


def _layer2_body(acc_ref, h1_ref, dinv_ref, b1_ref, w2_ref, h2_ref):
    dv = dinv_ref[...]
    z = jnp.maximum((acc_ref[...] + h1_ref[...]) * dv + b1_ref[...], 0.0)
    h2_ref[...] = jnp.dot(z, w2_ref[...],
                          preferred_element_type=jnp.float32) * dv


def _final_body(acc_ref, h2_ref, dinv_ref, b2_ref, out_ref):
    out_ref[...] = ((acc_ref[...] + h2_ref[...]) * dinv_ref[...]
                    + b2_ref[...])


_row_spec = pl.BlockSpec((_BM, D), lambda i: (i, 0))
_col_spec = pl.BlockSpec((_BM, 1), lambda i: (i, 0))
_w_spec = pl.BlockSpec((D, D), lambda i: (0, 0))
_b_spec = pl.BlockSpec((1, D), lambda i: (0, 0))

_mm_scale = pl.pallas_call(
    _mm_scale_body,
    grid=(_GRID,),
    in_specs=[_row_spec, _w_spec, _col_spec],
    out_specs=[_row_spec, _col_spec],
    out_shape=[jax.ShapeDtypeStruct((N, D), jnp.float32),
               jax.ShapeDtypeStruct((N, 1), jnp.float32)],
)

_layer2 = pl.pallas_call(
    _layer2_body,
    grid=(_GRID,),
    in_specs=[_row_spec, _row_spec, _col_spec, _b_spec, _w_spec],
    out_specs=_row_spec,
    out_shape=jax.ShapeDtypeStruct((N, D), jnp.float32),
)

_final = pl.pallas_call(
    _final_body,
    grid=(_GRID,),
    in_specs=[_row_spec, _row_spec, _col_spec, _b_spec],
    out_specs=_row_spec,
    out_shape=jax.ShapeDtypeStruct((N, D), jnp.float32),
)


def kernel(x, edge_index, W1, b1, W2, b2):
    src = edge_index[0]
    dst = edge_index[1]

    degp = _deg_kernel(dst).reshape(NW, NP)
    deg = _deg_reduce(degp)[:N]
    qp_h, cnt_h = _prep_kernel(src, dst)
    h1, dinv = _mm_scale(x, W1, deg)
    acc1 = _agg_kernel(h1, qp_h, cnt_h)[:N]
    h2 = _layer2(acc1, h1, dinv, b1.reshape(1, D), W2)
    acc2 = _agg_kernel(h2, qp_h, cnt_h)[:N]
    out = _final(acc2, h2, dinv, b2.reshape(1, D))
    return out
